# baseline (speedup 1.0000x reference)
import functools

import jax
import jax.numpy as jnp
from jax.experimental import pallas as pl

H = 128


def _cell_kernel(ohi_ref, olo_ref, x3_ref, cc_ref, ahi_ref, alo_ref, b_ref,
                 h_ref, c_ref):
    g = jnp.dot(x3_ref[...], b_ref[...], preferred_element_type=jnp.float32)
    o_hi = ohi_ref[...]
    o_lo = olo_ref[...]
    a_hi = ahi_ref[...]
    a_lo = alo_ref[...]
    g = g + jnp.dot(o_hi, a_hi, preferred_element_type=jnp.float32)
    g = g + jnp.dot(o_hi, a_lo, preferred_element_type=jnp.float32)
    g = g + jnp.dot(o_lo, a_hi, preferred_element_type=jnp.float32)
    cc = cc_ref[...]
    f1 = jax.nn.sigmoid(g[:, 0:H])
    f2 = jax.nn.sigmoid(g[:, H:2 * H])
    c_children = f1 * cc[:, :H] + f2 * cc[:, H:]
    i = jax.nn.sigmoid(g[:, 2 * H:3 * H])
    o = jax.nn.sigmoid(g[:, 3 * H:4 * H])
    u = jnp.tanh(g[:, 4 * H:5 * H])
    c = i * u + c_children
    h_ref[...] = o * jnp.tanh(c)
    c_ref[...] = c


def kernel(x, child_h, child_c, A_f, U1_f, U2_f, b_f, A_iou, U1_iou, U2_iou, b_iou_agg, W_iou, b_iou):
    n = x.shape[0]
    bn = 80 if n % 80 == 0 else 8
    grid = n // bn

    h1 = child_h[:, 0, :]
    h2 = child_h[:, 1, :]

    outer = (h1[:, :, None] * h2[:, None, :]).reshape(n, H * H)
    o_hi = outer.astype(jnp.bfloat16)
    o_lo = (outer - o_hi.astype(jnp.float32)).astype(jnp.bfloat16)

    ones = jnp.ones((n, 1), dtype=jnp.float32)
    zeros = jnp.zeros((n, H - 1), dtype=jnp.float32)
    x3 = jnp.concatenate([h1, h2, x, ones, zeros], axis=1).astype(jnp.bfloat16)

    a_all = jnp.concatenate(
        [A_f.reshape(H * H, 2 * H), A_iou.reshape(H * H, 3 * H)], axis=1)
    a_hi = a_all.astype(jnp.bfloat16)
    a_lo = (a_all - a_hi.astype(jnp.float32)).astype(jnp.bfloat16)

    u1 = jnp.concatenate([U1_f, U1_iou], axis=1)
    u2 = jnp.concatenate([U2_f, U2_iou], axis=1)
    w = jnp.concatenate([jnp.zeros((H, 2 * H), x.dtype), W_iou], axis=1)
    bias = jnp.concatenate([b_f, b_iou_agg + b_iou[0]])[None, :]
    b_small = jnp.concatenate(
        [u1, u2, w, bias, jnp.zeros((H - 1, 5 * H), x.dtype)], axis=0
    ).astype(jnp.bfloat16)

    cc = child_c.reshape(n, 2 * H)

    h_out, c_out = pl.pallas_call(
        _cell_kernel,
        grid=(grid,),
        in_specs=[
            pl.BlockSpec((bn, H * H), lambda i: (i, 0)),
            pl.BlockSpec((bn, H * H), lambda i: (i, 0)),
            pl.BlockSpec((bn, 4 * H), lambda i: (i, 0)),
            pl.BlockSpec((bn, 2 * H), lambda i: (i, 0)),
            pl.BlockSpec((H * H, 5 * H), lambda i: (0, 0)),
            pl.BlockSpec((H * H, 5 * H), lambda i: (0, 0)),
            pl.BlockSpec((4 * H, 5 * H), lambda i: (0, 0)),
        ],
        out_specs=[
            pl.BlockSpec((bn, H), lambda i: (i, 0)),
            pl.BlockSpec((bn, H), lambda i: (i, 0)),
        ],
        out_shape=[
            jax.ShapeDtypeStruct((n, H), jnp.float32),
            jax.ShapeDtypeStruct((n, H), jnp.float32),
        ],
    )(o_hi, o_lo, x3, cc, a_hi, a_lo, b_small)
    return (h_out, c_out)


# speedup vs baseline: 1.2625x; 1.2625x over previous
import functools

import jax
import jax.numpy as jnp
from jax.experimental import pallas as pl
from jax.experimental.pallas import tpu as pltpu

H = 128
KC = 2048
NKC = (H * H) // KC


def _cell_kernel(ohi_ref, olo_ref, x3_ref, cc_ref, ahi_ref, alo_ref, b_ref,
                 h_ref, c_ref, g_ref):
    kc = pl.program_id(1)

    @pl.when(kc == 0)
    def _init():
        g_ref[...] = jnp.dot(x3_ref[...], b_ref[...],
                             preferred_element_type=jnp.float32)

    o_hi = ohi_ref[...]
    o_lo = olo_ref[...]
    a_hi = ahi_ref[...]
    a_lo = alo_ref[...]
    g_ref[...] += (
        jnp.dot(o_hi, a_hi, preferred_element_type=jnp.float32)
        + jnp.dot(o_hi, a_lo, preferred_element_type=jnp.float32)
        + jnp.dot(o_lo, a_hi, preferred_element_type=jnp.float32))

    @pl.when(kc == NKC - 1)
    def _tail():
        g = g_ref[...]
        cc = cc_ref[...]
        f1 = jax.nn.sigmoid(g[:, 0:H])
        f2 = jax.nn.sigmoid(g[:, H:2 * H])
        c_children = f1 * cc[:, :H] + f2 * cc[:, H:]
        i = jax.nn.sigmoid(g[:, 2 * H:3 * H])
        o = jax.nn.sigmoid(g[:, 3 * H:4 * H])
        u = jnp.tanh(g[:, 4 * H:5 * H])
        c = i * u + c_children
        h_ref[...] = o * jnp.tanh(c)
        c_ref[...] = c


def kernel(x, child_h, child_c, A_f, U1_f, U2_f, b_f, A_iou, U1_iou, U2_iou, b_iou_agg, W_iou, b_iou):
    n = x.shape[0]
    bn = 1000 if n % 1000 == 0 else 8
    grid = (n // bn, NKC)

    h1 = child_h[:, 0, :]
    h2 = child_h[:, 1, :]

    outer = (h1[:, :, None] * h2[:, None, :]).reshape(n, H * H)
    o_hi = outer.astype(jnp.bfloat16)
    o_lo = (outer - o_hi.astype(jnp.float32)).astype(jnp.bfloat16)

    ones = jnp.ones((n, 1), dtype=jnp.float32)
    zeros = jnp.zeros((n, H - 1), dtype=jnp.float32)
    x3 = jnp.concatenate([h1, h2, x, ones, zeros], axis=1).astype(jnp.bfloat16)

    a_all = jnp.concatenate(
        [A_f.reshape(H * H, 2 * H), A_iou.reshape(H * H, 3 * H)], axis=1)
    a_hi = a_all.astype(jnp.bfloat16)
    a_lo = (a_all - a_hi.astype(jnp.float32)).astype(jnp.bfloat16)

    u1 = jnp.concatenate([U1_f, U1_iou], axis=1)
    u2 = jnp.concatenate([U2_f, U2_iou], axis=1)
    w = jnp.concatenate([jnp.zeros((H, 2 * H), x.dtype), W_iou], axis=1)
    bias = jnp.concatenate([b_f, b_iou_agg + b_iou[0]])[None, :]
    b_small = jnp.concatenate(
        [u1, u2, w, bias, jnp.zeros((H - 1, 5 * H), x.dtype)], axis=0
    ).astype(jnp.bfloat16)

    cc = child_c.reshape(n, 2 * H)

    h_out, c_out = pl.pallas_call(
        _cell_kernel,
        grid=grid,
        in_specs=[
            pl.BlockSpec((bn, KC), lambda i, k: (i, k)),
            pl.BlockSpec((bn, KC), lambda i, k: (i, k)),
            pl.BlockSpec((bn, 4 * H), lambda i, k: (i, 0)),
            pl.BlockSpec((bn, 2 * H), lambda i, k: (i, 0)),
            pl.BlockSpec((KC, 5 * H), lambda i, k: (k, 0)),
            pl.BlockSpec((KC, 5 * H), lambda i, k: (k, 0)),
            pl.BlockSpec((4 * H, 5 * H), lambda i, k: (0, 0)),
        ],
        out_specs=[
            pl.BlockSpec((bn, H), lambda i, k: (i, 0)),
            pl.BlockSpec((bn, H), lambda i, k: (i, 0)),
        ],
        out_shape=[
            jax.ShapeDtypeStruct((n, H), jnp.float32),
            jax.ShapeDtypeStruct((n, H), jnp.float32),
        ],
        scratch_shapes=[pltpu.VMEM((bn, 5 * H), jnp.float32)],
    )(o_hi, o_lo, x3, cc, a_hi, a_lo, b_small)
    return (h_out, c_out)


# speedup vs baseline: 4.0233x; 3.1868x over previous
import functools

import jax
import jax.numpy as jnp
from jax.experimental import pallas as pl

H = 128
CH = 16


def _cell_kernel(x3_ref, cc_ref, ahi_ref, alo_ref, b_ref, h_ref, c_ref, *, bn):
    x3 = x3_ref[...]
    h1 = x3[:, :H]
    h2 = x3[:, H:2 * H]
    g = jnp.dot(x3.astype(jnp.bfloat16), b_ref[...],
                preferred_element_type=jnp.float32)
    mean_term = 0.5 * (jnp.sum(h1, axis=1, keepdims=True)
                       * jnp.sum(h2, axis=1, keepdims=True))
    g = g + mean_term
    for t in range(H // CH):
        chunk = (h1[:, t * CH:(t + 1) * CH, None] * h2[:, None, :]).reshape(
            bn, CH * H)
        o_hi = chunk.astype(jnp.bfloat16)
        a_hi = ahi_ref[t * CH * H:(t + 1) * CH * H, :]
        a_lo = alo_ref[t * CH * H:(t + 1) * CH * H, :]
        g = g + jnp.dot(o_hi, a_hi, preferred_element_type=jnp.float32)
        g = g + jnp.dot(o_hi, a_lo, preferred_element_type=jnp.float32)
    cc = cc_ref[...]
    f1 = jax.nn.sigmoid(g[:, 0:H])
    f2 = jax.nn.sigmoid(g[:, H:2 * H])
    c_children = f1 * cc[:, :H] + f2 * cc[:, H:]
    i = jax.nn.sigmoid(g[:, 2 * H:3 * H])
    o = jax.nn.sigmoid(g[:, 3 * H:4 * H])
    u = jnp.tanh(g[:, 4 * H:5 * H])
    c = i * u + c_children
    h_ref[...] = o * jnp.tanh(c)
    c_ref[...] = c


def kernel(x, child_h, child_c, A_f, U1_f, U2_f, b_f, A_iou, U1_iou, U2_iou, b_iou_agg, W_iou, b_iou):
    n = x.shape[0]
    bn = 200 if n % 200 == 0 else 8
    grid = n // bn

    h1 = child_h[:, 0, :]
    h2 = child_h[:, 1, :]
    ones = jnp.ones((n, 1), dtype=jnp.float32)
    zeros = jnp.zeros((n, H - 1), dtype=jnp.float32)
    x3 = jnp.concatenate([h1, h2, x, ones, zeros], axis=1)

    a_all = jnp.concatenate(
        [A_f.reshape(H * H, 2 * H), A_iou.reshape(H * H, 3 * H)], axis=1) - 0.5
    a_hi = a_all.astype(jnp.bfloat16)
    a_lo = (a_all - a_hi.astype(jnp.float32)).astype(jnp.bfloat16)

    u1 = jnp.concatenate([U1_f, U1_iou], axis=1)
    u2 = jnp.concatenate([U2_f, U2_iou], axis=1)
    w = jnp.concatenate([jnp.zeros((H, 2 * H), x.dtype), W_iou], axis=1)
    bias = jnp.concatenate([b_f, b_iou_agg + b_iou[0]])[None, :]
    b_small = jnp.concatenate(
        [u1, u2, w, bias, jnp.zeros((H - 1, 5 * H), x.dtype)], axis=0
    ).astype(jnp.bfloat16)

    cc = child_c.reshape(n, 2 * H)

    h_out, c_out = pl.pallas_call(
        functools.partial(_cell_kernel, bn=bn),
        grid=(grid,),
        in_specs=[
            pl.BlockSpec((bn, 4 * H), lambda i: (i, 0)),
            pl.BlockSpec((bn, 2 * H), lambda i: (i, 0)),
            pl.BlockSpec((H * H, 5 * H), lambda i: (0, 0)),
            pl.BlockSpec((H * H, 5 * H), lambda i: (0, 0)),
            pl.BlockSpec((4 * H, 5 * H), lambda i: (0, 0)),
        ],
        out_specs=[
            pl.BlockSpec((bn, H), lambda i: (i, 0)),
            pl.BlockSpec((bn, H), lambda i: (i, 0)),
        ],
        out_shape=[
            jax.ShapeDtypeStruct((n, H), jnp.float32),
            jax.ShapeDtypeStruct((n, H), jnp.float32),
        ],
    )(x3, cc, a_hi, a_lo, b_small)
    return (h_out, c_out)
